# baseline (speedup 1.0000x reference)
import functools

import jax
import jax.numpy as jnp
from jax import lax
from jax.experimental import pallas as pl
from jax.experimental.pallas import tpu as pltpu
from jax.experimental.pallas import tpu_sc as plsc

TOP_K = 200
L = 16
NBINS = 8192
CAP = 512
CANDBUF = CAP + L
SELBUF = 256
H_UNROLL = 8
P2_UNROLL = 4


def _topk_pool_kernel(B, V, D, tfidf_arr, embedding):
    NW = 32
    RPW = B // NW
    NV = V // L
    DV = D // L
    mesh = plsc.VectorSubcoreMesh(core_axis_name="c", subcore_axis_name="s")

    @functools.partial(
        pl.kernel,
        mesh=mesh,
        out_type=jax.ShapeDtypeStruct((B, D), jnp.float32),
        compiler_params=pltpu.CompilerParams(needs_layout_passes=False),
        scratch_types=[
            pltpu.VMEM((V,), jnp.float32),
            pltpu.VMEM((NBINS,), jnp.float32),
            pltpu.VMEM((CANDBUF,), jnp.float32),
            pltpu.VMEM((CANDBUF,), jnp.int32),
            pltpu.VMEM((SELBUF,), jnp.float32),
            pltpu.VMEM((SELBUF,), jnp.int32),
            pltpu.VMEM((2, 128), jnp.int32),
            pltpu.VMEM((128, D), jnp.float32),
            pltpu.SemaphoreType.DMA,
        ],
    )
    def body(tf_hbm, emb_hbm, out_hbm, row_v, hist_v, cval_v, cidx_v,
             selw_v, self_v, selg_v, rows_v, sem):
        wid = lax.axis_index("s") * 2 + lax.axis_index("c")
        kf = jnp.float32(TOP_K)
        ones = jnp.ones((L,), jnp.float32)
        neg1 = jnp.full((L,), -1.0, jnp.float32)
        zeros_f = jnp.zeros((L,), jnp.float32)
        zeros_i = jnp.zeros((L,), jnp.int32)
        lane_iota = lax.iota(jnp.int32, L)
        scale = jnp.float32(NBINS)

        def bin_of(v):
            b = (v * scale).astype(jnp.int32)
            return jnp.minimum(jnp.maximum(b, 0), NBINS - 1)

        def do_row(r_local, carry):
            r = wid * RPW + r_local
            with jax.named_scope('ph_dma_row'):
                pltpu.sync_copy(tf_hbm.at[r], row_v)

            def zero_hist(i, c):
                hist_v[pl.ds(i * L, L)] = zeros_f
                return c
            with jax.named_scope('ph_zero_hist'):
                lax.fori_loop(0, NBINS // L, zero_hist, 0)

            def hist_step(i, c):
                for u in range(H_UNROLL):
                    v = row_v[pl.ds((i * H_UNROLL + u) * L, L)]
                    plsc.addupdate_scatter(hist_v, [bin_of(v)], ones)
                return c
            with jax.named_scope('ph_hist'):
                lax.fori_loop(0, NV // H_UNROLL, hist_step, 0)
                for q in range((NV // H_UNROLL) * H_UNROLL, NV):
                    v = row_v[pl.ds(q * L, L)]
                    plsc.addupdate_scatter(hist_v, [bin_of(v)], ones)

            def wcond(st):
                _, above = st
                return above < kf

            def wbody(st):
                j, above = st
                s = jnp.sum(hist_v[pl.ds(j * L, L)])
                return (j - 1, above + s)

            with jax.named_scope('ph_scan'):
                jend, above_end = lax.while_loop(
                    wcond, wbody, (jnp.int32(NBINS // L - 1), jnp.float32(0.0)))
            jc = jend + 1
            h = hist_v[pl.ds(jc * L, L)]
            above_prev = above_end - jnp.sum(h)
            sfx = lax.rev(plsc.cumsum(lax.rev(h, (0,))), (0,))
            in_top = (above_prev + sfx) >= kf
            b_star = jc * L + jnp.sum(in_top.astype(jnp.int32)) - 1

            def fill_cand(i, c):
                cval_v[pl.ds(i * L, L)] = neg1
                return c
            lax.fori_loop(0, CANDBUF // L, fill_cand, 0)

            def collect_one(q, ptr):
                v = row_v[pl.ds(q * L, L)]
                msk = bin_of(v) >= b_star
                ptr_c = jnp.minimum(ptr, CAP)
                plsc.store_compressed(cval_v.at[pl.ds(ptr_c, L)], v, mask=msk)
                plsc.store_compressed(
                    cidx_v.at[pl.ds(ptr_c, L)], q * L + lane_iota, mask=msk)
                return ptr_c + jnp.sum(msk.astype(jnp.int32))

            def p2_step(i, ptr):
                for u in range(P2_UNROLL):
                    ptr = collect_one(i * P2_UNROLL + u, ptr)
                return ptr
            with jax.named_scope('ph_collect'):
                ptr_main = lax.fori_loop(0, NV // P2_UNROLL, p2_step, jnp.int32(0))
                for q in range((NV // P2_UNROLL) * P2_UNROLL, NV):
                    ptr_main = collect_one(jnp.int32(q), ptr_main)

            def count_ge(t):
                def cg(q, acc):
                    bits = plsc.bitcast(cval_v[pl.ds(q * L, L)], jnp.int32)
                    return acc + jnp.sum((bits >= t).astype(jnp.int32))
                return lax.fori_loop(0, CANDBUF // L, cg, jnp.int32(0))

            def bs_step(_, st):
                lo, hi = st
                mid = lo + ((hi - lo + 1) >> 1)
                take = count_ge(mid) >= TOP_K
                return (jnp.where(take, mid, lo), jnp.where(take, hi, mid - 1))

            with jax.named_scope('ph_bsearch'):
                u_bits, _ = lax.fori_loop(
                    0, 31, bs_step, (jnp.int32(0), jnp.int32(0x7F7FFFFF)))
            n_gt = count_ge(u_bits + 1)
            n_tie_skip = (count_ge(u_bits) - n_gt) - (TOP_K - n_gt)

            def fill_sel(i, c):
                selw_v[pl.ds(i * L, L)] = zeros_f
                self_v[pl.ds(i * L, L)] = zeros_i
                return c
            lax.fori_loop(0, SELBUF // L, fill_sel, 0)

            def sel_step(q, st):
                ptr2, ties = st
                v = cval_v[pl.ds(q * L, L)]
                ids = cidx_v[pl.ds(q * L, L)]
                bits = plsc.bitcast(v, jnp.int32)
                gt = bits > u_bits
                tie = bits == u_bits
                trank = ties + plsc.cumsum(tie.astype(jnp.int32))
                inc = gt | (tie & (trank > n_tie_skip))
                plsc.store_compressed(selw_v.at[pl.ds(ptr2, L)], v, mask=inc)
                plsc.store_compressed(self_v.at[pl.ds(ptr2, L)], ids, mask=inc)
                return (ptr2 + jnp.sum(inc.astype(jnp.int32)),
                        ties + jnp.sum(tie.astype(jnp.int32)))
            with jax.named_scope('ph_select'):
                lax.fori_loop(0, CANDBUF // L, sel_step,
                              (jnp.int32(0), jnp.int32(0)))

                for chunk in range(2):
                    for col in range(128 // L):
                        selg_v[chunk, pl.ds(col * L, L)] = (
                            self_v[pl.ds(chunk * 128 + col * L, L)])

            accs = tuple(jnp.zeros((L,), jnp.float32) for _ in range(DV))
            with jax.named_scope('ph_gather_acc'):
                for chunk in range(2):
                    pltpu.async_copy(
                        emb_hbm.at[selg_v.at[chunk]], rows_v, sem).wait()

                    def acc_step(k2, a):
                        w = plsc.load_gather(
                            selw_v, [jnp.full((L,), chunk * 128 + k2, jnp.int32)])
                        return tuple(a[d] + w * rows_v[k2, pl.ds(d * L, L)]
                                     for d in range(DV))
                    accs = lax.fori_loop(0, 128, acc_step, accs)

            inv_k = jnp.float32(1.0 / TOP_K)
            for d in range(DV):
                selw_v[pl.ds(d * L, L)] = accs[d] * inv_k
            pltpu.sync_copy(selw_v.at[pl.ds(0, D)], out_hbm.at[r])
            return carry

        lax.fori_loop(0, RPW, do_row, 0)

    return body(tfidf_arr, embedding)


@jax.jit
def kernel(tfidf_arr, embedding):
    B, V = tfidf_arr.shape
    _, D = embedding.shape
    return _topk_pool_kernel(B, V, D, tfidf_arr, embedding)


# speedup vs baseline: 1.4848x; 1.4848x over previous
import functools

import jax
import jax.numpy as jnp
from jax import lax
from jax.experimental import pallas as pl
from jax.experimental.pallas import tpu as pltpu
from jax.experimental.pallas import tpu_sc as plsc

TOP_K = 200
L = 16
NBINS = 8192
CANDBUF = 528
SELBUF = 256
BLOCK = 10
P2_UNROLL = 5


def _topk_pool_kernel(B, V, D, tfidf_arr, embedding):
    NW = 32
    RPW = B // NW
    NV = V // L
    DV = D // L
    mesh = plsc.VectorSubcoreMesh(core_axis_name="c", subcore_axis_name="s")

    @functools.partial(
        pl.kernel,
        mesh=mesh,
        out_type=jax.ShapeDtypeStruct((B, D), jnp.float32),
        compiler_params=pltpu.CompilerParams(needs_layout_passes=False),
        scratch_types=[
            pltpu.VMEM((V,), jnp.float32),
            pltpu.VMEM((NBINS,), jnp.float32),
            pltpu.VMEM((CANDBUF,), jnp.int32),
            pltpu.VMEM((SELBUF,), jnp.float32),
            pltpu.VMEM((SELBUF,), jnp.int32),
            pltpu.VMEM((2, 128), jnp.int32),
            pltpu.VMEM((128, D), jnp.float32),
            pltpu.SemaphoreType.DMA,
        ],
    )
    def body(tf_hbm, emb_hbm, out_hbm, row_v, hist_v, cidx_v,
             selw_v, self_v, selg_v, rows_v, sem):
        wid = lax.axis_index("s") * 2 + lax.axis_index("c")
        kf = jnp.float32(TOP_K)
        k_spl = jnp.full((L,), TOP_K, jnp.int32)
        ones = jnp.ones((L,), jnp.float32)
        zeros_f = jnp.zeros((L,), jnp.float32)
        zeros_i = jnp.zeros((L,), jnp.int32)
        lane_iota = lax.iota(jnp.int32, L)
        scale = jnp.float32(NBINS)
        capclamp = jnp.full((L,), CANDBUF - 1, jnp.int32)

        def bin_of(v):
            b = (v * scale).astype(jnp.int32)
            return jnp.minimum(jnp.maximum(b, 0), NBINS - 1)

        def popcnt(mask):
            return plsc.all_reduce_population_count(mask)

        def zero_ids(i, c):
            cidx_v[pl.ds(i * L, L)] = zeros_i
            return c
        lax.fori_loop(0, CANDBUF // L, zero_ids, 0)

        def fill_sel(i, c):
            selw_v[pl.ds(i * L, L)] = zeros_f
            self_v[pl.ds(i * L, L)] = zeros_i
            return c
        lax.fori_loop(0, SELBUF // L, fill_sel, 0)

        def do_row(r_local, carry):
            r = wid * RPW + r_local
            with jax.named_scope('ph_dma_row'):
                pltpu.sync_copy(tf_hbm.at[r], row_v)

            def zero_hist(i, c):
                hist_v[pl.ds(i * L, L)] = zeros_f
                return c
            with jax.named_scope('ph_zero_hist'):
                lax.fori_loop(0, NBINS // L, zero_hist, 0)

            def hist_step(i, c):
                base = i * BLOCK
                m = row_v[pl.ds(base * L, L)]
                for u in range(1, BLOCK):
                    m = jnp.maximum(m, row_v[pl.ds((base + u) * L, L)])
                plsc.addupdate_scatter(hist_v, [bin_of(m)], ones)
                return c
            with jax.named_scope('ph_hist'):
                lax.fori_loop(0, NV // BLOCK, hist_step, 0)

            def wcond(st):
                _, above = st
                return above < kf

            def wbody(st):
                j, above = st
                s = jnp.sum(hist_v[pl.ds(j * L, L)])
                return (j - 1, above + s)

            with jax.named_scope('ph_scan'):
                jend, above_end = lax.while_loop(
                    wcond, wbody, (jnp.int32(NBINS // L - 1), jnp.float32(0.0)))
            jc = jend + 1
            h = hist_v[pl.ds(jc * L, L)]
            above_prev = above_end - jnp.sum(h)
            sfx = lax.rev(plsc.cumsum(lax.rev(h, (0,))), (0,))
            in_top = (above_prev + sfx) >= kf
            b_star = jc * L + jnp.sum(in_top.astype(jnp.int32)) - 1
            edge = jnp.full((L,), b_star.astype(jnp.float32)
                            * jnp.float32(1.0 / NBINS), jnp.float32)

            def collect_one(q, ptr):
                v = row_v[pl.ds(q * L, L)]
                msk = v >= edge
                rank = plsc.cumsum(msk.astype(jnp.int32))
                addr = jnp.minimum(ptr + rank - 1, capclamp)
                plsc.store_scatter(cidx_v, [addr], q * L + lane_iota, mask=msk)
                return ptr + popcnt(msk)

            def p2_step(i, ptr):
                for u in range(P2_UNROLL):
                    ptr = collect_one(i * P2_UNROLL + u, ptr)
                return ptr

            with jax.named_scope('ph_collect'):
                c_spl = lax.fori_loop(0, NV // P2_UNROLL, p2_step, zeros_i)
                for q in range((NV // P2_UNROLL) * P2_UNROLL, NV):
                    c_spl = collect_one(jnp.int32(q), c_spl)

            def count_ge(t_spl):
                def cg(q, acc):
                    pos = q * L + lane_iota
                    ids = cidx_v[pl.ds(q * L, L)]
                    vals = plsc.load_gather(row_v, [ids])
                    bits = plsc.bitcast(vals, jnp.int32)
                    ok = (bits >= t_spl) & (pos < c_spl)
                    return acc + popcnt(ok)
                return lax.fori_loop(0, CANDBUF // L, cg, zeros_i)

            def bs_step(_, st):
                lo, hi = st
                mid = lo + ((hi - lo + 1) >> 1)
                take = count_ge(mid) >= k_spl
                return (jnp.where(take, mid, lo), jnp.where(take, hi, mid - 1))

            with jax.named_scope('ph_bsearch'):
                u_bits, _ = lax.fori_loop(
                    0, 31, bs_step,
                    (zeros_i, jnp.full((L,), 0x7F7FFFFF, jnp.int32)))
                n_tie_skip = count_ge(u_bits) - k_spl

            def sel_step(q, st):
                ptr2, ties = st
                pos = q * L + lane_iota
                ids = cidx_v[pl.ds(q * L, L)]
                vals = plsc.load_gather(row_v, [ids])
                bits = plsc.bitcast(vals, jnp.int32)
                valid = pos < c_spl
                gt = (bits > u_bits) & valid
                tie = (bits == u_bits) & valid
                trank = ties + plsc.cumsum(tie.astype(jnp.int32))
                inc = gt | (tie & (trank > n_tie_skip))
                rank = plsc.cumsum(inc.astype(jnp.int32))
                addr = ptr2 + rank - 1
                plsc.store_scatter(selw_v, [addr], vals, mask=inc)
                plsc.store_scatter(self_v, [addr], ids, mask=inc)
                return (ptr2 + popcnt(inc), ties + popcnt(tie))

            with jax.named_scope('ph_select'):
                lax.fori_loop(0, CANDBUF // L, sel_step, (zeros_i, zeros_i))

                for chunk in range(2):
                    for col in range(128 // L):
                        selg_v[chunk, pl.ds(col * L, L)] = (
                            self_v[pl.ds(chunk * 128 + col * L, L)])

            accs = tuple(jnp.zeros((L,), jnp.float32) for _ in range(DV))
            with jax.named_scope('ph_gather_acc'):
                for chunk in range(2):
                    pltpu.async_copy(
                        emb_hbm.at[selg_v.at[chunk]], rows_v, sem).wait()

                    def acc_step(k2, a):
                        w = plsc.load_gather(
                            selw_v, [jnp.full((L,), chunk * 128 + k2, jnp.int32)])
                        return tuple(a[d] + w * rows_v[k2, pl.ds(d * L, L)]
                                     for d in range(DV))
                    accs = lax.fori_loop(0, 128, acc_step, accs)

            inv_k = jnp.float32(1.0 / TOP_K)
            for d in range(DV):
                selw_v[pl.ds(d * L, L)] = accs[d] * inv_k
            pltpu.sync_copy(selw_v.at[pl.ds(0, D)], out_hbm.at[r])
            return carry

        lax.fori_loop(0, RPW, do_row, 0)

    return body(tfidf_arr, embedding)


@jax.jit
def kernel(tfidf_arr, embedding):
    B, V = tfidf_arr.shape
    _, D = embedding.shape
    return _topk_pool_kernel(B, V, D, tfidf_arr, embedding)


# speedup vs baseline: 2.3668x; 1.5941x over previous
import functools

import jax
import jax.numpy as jnp
from jax import lax
from jax.experimental import pallas as pl
from jax.experimental.pallas import tpu as pltpu
from jax.experimental.pallas import tpu_sc as plsc

TOP_K = 200
L = 16
NBINS = 8192
CAP = 512
CANDBUF = CAP + L
SELBUF = 256
BLOCK = 25
ZH_UNROLL = 8
PB_UNROLL = 5
BLK_UNROLL = 5


def _topk_pool_kernel(B, V, D, tfidf_arr, embedding):
    NW = 32
    RPW = B // NW
    NV = V // L
    DV = D // L
    NBLK = NV // BLOCK
    mesh = plsc.VectorSubcoreMesh(core_axis_name="c", subcore_axis_name="s")

    @functools.partial(
        pl.kernel,
        mesh=mesh,
        out_type=jax.ShapeDtypeStruct((B, D), jnp.float32),
        compiler_params=pltpu.CompilerParams(needs_layout_passes=False),
        scratch_types=[
            pltpu.VMEM((V,), jnp.float32),
            pltpu.VMEM((NBINS,), jnp.float32),
            pltpu.VMEM((NBLK * L,), jnp.float32),
            pltpu.VMEM((CANDBUF,), jnp.int32),
            pltpu.VMEM((SELBUF,), jnp.float32),
            pltpu.VMEM((SELBUF,), jnp.int32),
            pltpu.VMEM((2, 128), jnp.int32),
            pltpu.VMEM((128, D), jnp.float32),
            pltpu.VMEM((D,), jnp.float32),
            pltpu.SemaphoreType.DMA,
            pltpu.SemaphoreType.DMA,
        ],
    )
    def body(tf_hbm, emb_hbm, out_hbm, row_v, hist_v, bmax_v, cidx_v,
             selw_v, self_v, selg_v, rows_v, outst_v, sem, sem_row):
        wid = lax.axis_index("s") * 2 + lax.axis_index("c")
        r0 = wid * RPW
        kf = jnp.float32(TOP_K)
        k_spl = jnp.full((L,), TOP_K, jnp.int32)
        ones = jnp.ones((L,), jnp.float32)
        zeros_f = jnp.zeros((L,), jnp.float32)
        zeros_i = jnp.zeros((L,), jnp.int32)
        lane_iota = lax.iota(jnp.int32, L)
        scale = jnp.float32(NBINS)

        def bin_of(v):
            b = (v * scale).astype(jnp.int32)
            return jnp.minimum(jnp.maximum(b, 0), NBINS - 1)

        def popcnt(mask):
            return plsc.all_reduce_population_count(mask)

        def zero_ids(i, c):
            cidx_v[pl.ds(i * L, L)] = zeros_i
            return c
        lax.fori_loop(0, CANDBUF // L, zero_ids, 0)

        def fill_sel(i, c):
            selw_v[pl.ds(i * L, L)] = zeros_f
            self_v[pl.ds(i * L, L)] = zeros_i
            return c
        lax.fori_loop(0, SELBUF // L, fill_sel, 0)

        pltpu.async_copy(tf_hbm.at[r0], row_v, sem_row)

        def do_row(r_local, carry):
            r = r0 + r_local
            with jax.named_scope('ph_dma_row'):
                pltpu.make_async_copy(tf_hbm.at[r], row_v, sem_row).wait()

            def zero_hist(i, c):
                for u in range(ZH_UNROLL):
                    hist_v[pl.ds((i * ZH_UNROLL + u) * L, L)] = zeros_f
                return c
            with jax.named_scope('ph_zero_hist'):
                lax.fori_loop(0, NBINS // (L * ZH_UNROLL), zero_hist, 0)

            def bmax_step(i, c):
                base = i * BLOCK
                m = row_v[pl.ds(base * L, L)]
                for u in range(1, BLOCK):
                    m = jnp.maximum(m, row_v[pl.ds((base + u) * L, L)])
                bmax_v[pl.ds(i * L, L)] = m
                return c
            with jax.named_scope('ph_bmax'):
                lax.fori_loop(0, NBLK, bmax_step, 0)

            def hist_step(i, c):
                for u in range(PB_UNROLL):
                    bm = bmax_v[pl.ds((i * PB_UNROLL + u) * L, L)]
                    plsc.addupdate_scatter(hist_v, [bin_of(bm)], ones)
                return c
            with jax.named_scope('ph_hist'):
                lax.fori_loop(0, NBLK // PB_UNROLL, hist_step, 0)

            def wcond(st):
                _, above = st
                return above < kf

            def wbody(st):
                j, above = st
                s = jnp.sum(hist_v[pl.ds(j * L, L)])
                return (j - 1, above + s)

            with jax.named_scope('ph_scan'):
                jend, above_end = lax.while_loop(
                    wcond, wbody, (jnp.int32(NBINS // L - 1), jnp.float32(0.0)))
            jc = jend + 1
            h = hist_v[pl.ds(jc * L, L)]
            above_prev = above_end - jnp.sum(h)
            sfx = lax.rev(plsc.cumsum(lax.rev(h, (0,))), (0,))
            in_top = (above_prev + sfx) >= kf
            b_star = jc * L + jnp.sum(in_top.astype(jnp.int32)) - 1
            edge = jnp.full((L,), b_star.astype(jnp.float32)
                            * jnp.float32(1.0 / NBINS), jnp.float32)

            def lane_cols(st):
                m, bi, p = st
                l_spl = plsc.all_reduce_ffs(m)
                base = bi * (BLOCK * L)
                idx1 = base + l_spl + lane_iota * L
                v1 = plsc.load_gather(row_v, [idx1])
                m1 = v1 >= edge
                pc = jnp.minimum(p, CAP)
                plsc.store_compressed(cidx_v.at[pl.ds(pc, L)], idx1, mask=m1)
                p = pc + jnp.sum(m1.astype(jnp.int32))
                idx2 = idx1 + L * L
                v2 = plsc.load_gather(row_v, [jnp.minimum(idx2, V - 1)])
                m2 = (v2 >= edge) & (lane_iota < (BLOCK - L))
                pc = jnp.minimum(p, CAP)
                plsc.store_compressed(cidx_v.at[pl.ds(pc, L)], idx2, mask=m2)
                p = pc + jnp.sum(m2.astype(jnp.int32))
                return (m & (lane_iota != l_spl), bi, p)

            def lane_cond(st):
                m, _, _ = st
                return jnp.any(m)

            def collect_blk(bi, ptr):
                flags = bmax_v[pl.ds(bi * L, L)] >= edge
                _, _, ptr = lax.while_loop(
                    lane_cond, lane_cols, (flags, bi, ptr))
                return ptr

            def collect_step(i, ptr):
                for u in range(BLK_UNROLL):
                    ptr = collect_blk(i * BLK_UNROLL + u, ptr)
                return ptr

            with jax.named_scope('ph_collect'):
                c_cnt = lax.fori_loop(0, NBLK // BLK_UNROLL, collect_step,
                                      jnp.int32(0))
                for q in range((NBLK // BLK_UNROLL) * BLK_UNROLL, NBLK):
                    c_cnt = collect_blk(jnp.int32(q), c_cnt)
            c_spl = jnp.full((L,), c_cnt, jnp.int32)
            nvc = (c_cnt + (L - 1)) >> 4

            def count_ge(t_spl):
                def cg(q, acc):
                    pos = q * L + lane_iota
                    ids = cidx_v[pl.ds(q * L, L)]
                    vals = plsc.load_gather(row_v, [ids])
                    bits = plsc.bitcast(vals, jnp.int32)
                    ok = (bits >= t_spl) & (pos < c_spl)
                    return acc + popcnt(ok)
                return lax.fori_loop(0, nvc, cg, zeros_i)

            def bs_step(_, st):
                lo, hi = st
                mid = lo + ((hi - lo + 1) >> 1)
                take = count_ge(mid) >= k_spl
                return (jnp.where(take, mid, lo), jnp.where(take, hi, mid - 1))

            with jax.named_scope('ph_bsearch'):
                u_bits, _ = lax.fori_loop(
                    0, 31, bs_step,
                    (zeros_i, jnp.full((L,), 0x7F7FFFFF, jnp.int32)))
                n_tie_skip = count_ge(u_bits) - k_spl

            def sel_step(q, st):
                ptr2, ties = st
                pos = q * L + lane_iota
                ids = cidx_v[pl.ds(q * L, L)]
                vals = plsc.load_gather(row_v, [ids])
                bits = plsc.bitcast(vals, jnp.int32)
                valid = pos < c_spl
                gt = (bits > u_bits) & valid
                tie = (bits == u_bits) & valid
                trank = ties + plsc.cumsum(tie.astype(jnp.int32))
                inc = gt | (tie & (trank > n_tie_skip))
                rank = plsc.cumsum(inc.astype(jnp.int32))
                addr = ptr2 + rank - 1
                plsc.store_scatter(selw_v, [addr], vals, mask=inc)
                plsc.store_scatter(self_v, [addr], ids, mask=inc)
                return (ptr2 + popcnt(inc), ties + popcnt(tie))

            with jax.named_scope('ph_select'):
                lax.fori_loop(0, nvc, sel_step, (zeros_i, zeros_i))

                for chunk in range(2):
                    for col in range(128 // L):
                        selg_v[chunk, pl.ds(col * L, L)] = (
                            self_v[pl.ds(chunk * 128 + col * L, L)])

            rnext = jnp.minimum(r + 1, r0 + RPW - 1)
            pltpu.async_copy(tf_hbm.at[rnext], row_v, sem_row)

            accs = tuple(jnp.zeros((L,), jnp.float32) for _ in range(DV))
            with jax.named_scope('ph_gather_acc'):
                for chunk in range(2):
                    pltpu.async_copy(
                        emb_hbm.at[selg_v.at[chunk]], rows_v, sem).wait()

                    def acc_step(k2, a):
                        w = plsc.load_gather(
                            selw_v, [jnp.full((L,), chunk * 128 + k2, jnp.int32)])
                        return tuple(a[d] + w * rows_v[k2, pl.ds(d * L, L)]
                                     for d in range(DV))
                    accs = lax.fori_loop(0, 128, acc_step, accs)

            inv_k = jnp.float32(1.0 / TOP_K)
            with jax.named_scope('ph_out'):
                for d in range(DV):
                    outst_v[pl.ds(d * L, L)] = accs[d] * inv_k
                pltpu.sync_copy(outst_v, out_hbm.at[r])
            return carry

        lax.fori_loop(0, RPW, do_row, 0)
        pltpu.make_async_copy(tf_hbm.at[r0], row_v, sem_row).wait()

    return body(tfidf_arr, embedding)


@jax.jit
def kernel(tfidf_arr, embedding):
    B, V = tfidf_arr.shape
    _, D = embedding.shape
    return _topk_pool_kernel(B, V, D, tfidf_arr, embedding)
